# baseline (speedup 1.0000x reference)
import functools

import jax
import jax.numpy as jnp
from jax import lax
from jax.experimental import pallas as pl
from jax.experimental.pallas import tpu as pltpu
from jax.experimental.pallas import tpu_sc as plsc

N = 10000
E = 320000
D = 128
H = 128

NC = 2
NS = 16
NW = NC * NS
CH = 128
EPAD = 2560 * CH
ROWS = EPAD // CH
RPW = ROWS // NW
RPS = ROWS // NS
NPAD = 10240
NPS = NPAD // NS
NHALF = NPAD // NC
TRASH = 128
ACC = NHALF + TRASH
APS = ACC // NS
DW = 128

_MESH = plsc.VectorSubcoreMesh(
    core_axis_name="c", subcore_axis_name="s", num_cores=NC, num_subcores=NS)


@functools.partial(
    pl.kernel,
    out_type=jax.ShapeDtypeStruct((NC, NHALF, DW), jnp.float32),
    mesh=_MESH,
    scratch_types=[
        pltpu.VMEM((RPS, CH), jnp.int32),
        pltpu.VMEM((CH, DW), jnp.float32),
        pltpu.VMEM_SHARED((ACC, DW), jnp.float32),
    ],
)
def _deg_sc(dstl_hbm, ones_hbm, zeros_hbm, out_hbm, idx_v, ones_v, acc):
    cid = lax.axis_index("c")
    sid = lax.axis_index("s")
    pltpu.sync_copy(zeros_hbm, acc.at[pl.ds(sid * APS, APS)])
    pltpu.sync_copy(ones_hbm, ones_v)
    pltpu.sync_copy(dstl_hbm.at[cid, pl.ds(sid * RPS, RPS)], idx_v)
    plsc.subcore_barrier()

    def body(c, carry):
        pltpu.sync_copy(ones_v, acc.at[idx_v.at[c]], add=True)
        return carry

    lax.fori_loop(0, RPS, body, 0)
    plsc.subcore_barrier()
    pltpu.sync_copy(acc.at[pl.ds(sid * (NHALF // NS), NHALF // NS)],
                    out_hbm.at[cid, pl.ds(sid * (NHALF // NS), NHALF // NS)])


@functools.partial(
    pl.kernel,
    out_type=jax.ShapeDtypeStruct((NC, NHALF, D), jnp.float32),
    mesh=_MESH,
    scratch_types=[
        pltpu.VMEM((RPS, CH), jnp.int32),
        pltpu.VMEM((RPS, CH), jnp.int32),
        pltpu.VMEM((2, CH, D), jnp.float32),
        pltpu.VMEM_SHARED((ACC, D), jnp.float32),
        pltpu.SemaphoreType.DMA,
        pltpu.SemaphoreType.DMA,
        pltpu.SemaphoreType.DMA,
        pltpu.SemaphoreType.DMA,
    ],
)
def _agg_sc(g_hbm, src_hbm, dstl_hbm, zeros_hbm, out_hbm,
            src_v, dst_v, buf, acc, g0, g1, s0, s1):
    cid = lax.axis_index("c")
    sid = lax.axis_index("s")
    gsem = (g0, g1)
    ssem = (s0, s1)
    pltpu.sync_copy(zeros_hbm, acc.at[pl.ds(sid * APS, APS)])
    pltpu.sync_copy(src_hbm.at[pl.ds(sid * RPS, RPS)], src_v)
    pltpu.sync_copy(dstl_hbm.at[cid, pl.ds(sid * RPS, RPS)], dst_v)
    plsc.subcore_barrier()

    pltpu.async_copy(g_hbm.at[src_v.at[0]], buf.at[0], gsem[0])

    def body(i, carry):
        for b in range(2):
            c = 2 * i + b
            bo = 1 - b
            pltpu.make_async_copy(
                g_hbm.at[src_v.at[c]], buf.at[b], gsem[b]).wait()
            pltpu.async_copy(buf.at[b], acc.at[dst_v.at[c]], ssem[b], add=True)

            @pl.when(c >= 1)
            def _():
                pltpu.make_async_copy(
                    buf.at[bo], acc.at[dst_v.at[c - 1]], ssem[bo]).wait()

            @pl.when(c + 1 < RPS)
            def _():
                pltpu.async_copy(
                    g_hbm.at[src_v.at[c + 1]], buf.at[bo], gsem[bo])
        return carry

    lax.fori_loop(0, RPS // 2, body, 0)
    pltpu.make_async_copy(
        buf.at[1], acc.at[dst_v.at[RPS - 1]], ssem[1]).wait()
    plsc.subcore_barrier()
    pltpu.sync_copy(acc.at[pl.ds(sid * (NHALF // NS), NHALF // NS)],
                    out_hbm.at[cid, pl.ds(sid * (NHALF // NS), NHALF // NS)])


BN = 1000


def _dinv_of(dp_ref):
    return lax.rsqrt(dp_ref[:, 0] + 1.0)


def _dstl_body(dst_ref, out_ref):
    d = dst_ref[...]
    for c in range(NC):
        base = c * NHALF
        inr = jnp.logical_and(d >= base, d < base + NHALF)
        out_ref[c] = jnp.where(inr, d - base, NHALF + (d & (TRASH - 1)))


def _dstl(dst):
    return pl.pallas_call(
        _dstl_body,
        in_specs=[pl.BlockSpec((ROWS, CH), lambda: (0, 0))],
        out_specs=pl.BlockSpec((NC, ROWS, CH), lambda: (0, 0, 0)),
        out_shape=jax.ShapeDtypeStruct((NC, ROWS, CH), jnp.int32),
    )(dst)


def _mm1_body(x_ref, w_ref, dp_ref, g_ref):
    dinv = _dinv_of(dp_ref)
    h = jnp.dot(x_ref[...], w_ref[...], preferred_element_type=jnp.float32)
    g_ref[...] = h * dinv[:, None]


def _mm2_body(s_ref, g_ref, dp_ref, b_ref, w_ref, out_ref):
    dinv = _dinv_of(dp_ref)
    s = s_ref[...] + g_ref[...]
    h1 = jnp.maximum(s * dinv[:, None] + b_ref[...], 0.0)
    h2 = jnp.dot(h1, w_ref[...], preferred_element_type=jnp.float32)
    out_ref[...] = h2 * dinv[:, None]


def _fin_body(s_ref, g_ref, dp_ref, b_ref, out_ref):
    dinv = _dinv_of(dp_ref)
    out_ref[...] = (s_ref[...] + g_ref[...]) * dinv[:, None] + b_ref[...]


def _mm1(x, w1, degp):
    return pl.pallas_call(
        _mm1_body,
        grid=(N // BN,),
        in_specs=[
            pl.BlockSpec((BN, D), lambda i: (i, 0)),
            pl.BlockSpec((D, H), lambda i: (0, 0)),
            pl.BlockSpec((BN, DW), lambda i: (i, 0)),
        ],
        out_specs=pl.BlockSpec((BN, H), lambda i: (i, 0)),
        out_shape=jax.ShapeDtypeStruct((N, H), jnp.float32),
    )(x, w1, degp)


def _mm2(s1, g1, degp, b1, w2):
    return pl.pallas_call(
        _mm2_body,
        grid=(N // BN,),
        in_specs=[
            pl.BlockSpec((BN, H), lambda i: (i, 0)),
            pl.BlockSpec((BN, H), lambda i: (i, 0)),
            pl.BlockSpec((BN, DW), lambda i: (i, 0)),
            pl.BlockSpec((1, H), lambda i: (0, 0)),
            pl.BlockSpec((H, H), lambda i: (0, 0)),
        ],
        out_specs=pl.BlockSpec((BN, H), lambda i: (i, 0)),
        out_shape=jax.ShapeDtypeStruct((N, H), jnp.float32),
    )(s1, g1, degp, b1, w2)


def _fin(s2, g2, degp, b2):
    return pl.pallas_call(
        _fin_body,
        grid=(N // BN,),
        in_specs=[
            pl.BlockSpec((BN, H), lambda i: (i, 0)),
            pl.BlockSpec((BN, H), lambda i: (i, 0)),
            pl.BlockSpec((BN, DW), lambda i: (i, 0)),
            pl.BlockSpec((1, H), lambda i: (0, 0)),
        ],
        out_specs=pl.BlockSpec((BN, H), lambda i: (i, 0)),
        out_shape=jax.ShapeDtypeStruct((N, H), jnp.float32),
    )(s2, g2, degp, b2)


def kernel(x, edge_index, W1, b1, W2, b2):
    ei = edge_index.astype(jnp.int32)
    pad = EPAD - E
    src = jnp.concatenate([ei[0], jnp.zeros((pad,), jnp.int32)])
    dst = jnp.concatenate([ei[1], jnp.full((pad,), N, jnp.int32)])
    src = src.reshape(ROWS, CH)
    dst = dst.reshape(ROWS, CH)
    ones_w = jnp.ones((CH, DW), jnp.float32)
    zeros_w = jnp.zeros((APS, DW), jnp.float32)
    zeros_rows = jnp.zeros((APS, D), jnp.float32)

    dstl = _dstl(dst)
    degp = _deg_sc(dstl, ones_w, zeros_w).reshape(NPAD, DW)[:N]
    g1 = _mm1(x, W1, degp)
    s1 = _agg_sc(g1, src, dstl, zeros_rows).reshape(NPAD, D)[:N]
    g2 = _mm2(s1, g1, degp, b1.reshape(1, H), W2)
    s2 = _agg_sc(g2, src, dstl, zeros_rows).reshape(NPAD, D)[:N]
    return _fin(s2, g2, degp, b2.reshape(1, H))


# speedup vs baseline: 1.0349x; 1.0349x over previous
import functools

import jax
import jax.numpy as jnp
from jax import lax
from jax.experimental import pallas as pl
from jax.experimental.pallas import tpu as pltpu
from jax.experimental.pallas import tpu_sc as plsc

N = 10000
E = 320000
D = 128
H = 128

NC = 2
NS = 16
NW = NC * NS
CH = 128
EPAD = 2560 * CH
ROWS = EPAD // CH
RPW = ROWS // NW
RPS = ROWS // NS
NPAD = 10240
NPS = NPAD // NS
NHALF = NPAD // NC
TRASH = 128
ACC = NHALF + TRASH
APS = ACC // NS
DW = 128

_MESH = plsc.VectorSubcoreMesh(
    core_axis_name="c", subcore_axis_name="s", num_cores=NC, num_subcores=NS)


@functools.partial(
    pl.kernel,
    out_type=jax.ShapeDtypeStruct((NC, NHALF, DW), jnp.float32),
    mesh=_MESH,
    scratch_types=[
        pltpu.VMEM((RPS, CH), jnp.int32),
        pltpu.VMEM((CH, DW), jnp.float32),
        pltpu.VMEM_SHARED((ACC, DW), jnp.float32),
    ],
)
def _deg_sc(dstl_hbm, ones_hbm, zeros_hbm, out_hbm, idx_v, ones_v, acc):
    cid = lax.axis_index("c")
    sid = lax.axis_index("s")
    pltpu.sync_copy(zeros_hbm, acc.at[pl.ds(sid * APS, APS)])
    pltpu.sync_copy(ones_hbm, ones_v)
    pltpu.sync_copy(dstl_hbm.at[cid, pl.ds(sid * RPS, RPS)], idx_v)
    plsc.subcore_barrier()

    def body(c, carry):
        pltpu.sync_copy(ones_v, acc.at[idx_v.at[c]], add=True)
        return carry

    lax.fori_loop(0, RPS, body, 0)
    plsc.subcore_barrier()
    pltpu.sync_copy(acc.at[pl.ds(sid * (NHALF // NS), NHALF // NS)],
                    out_hbm.at[cid, pl.ds(sid * (NHALF // NS), NHALF // NS)])


@functools.partial(
    pl.kernel,
    out_type=jax.ShapeDtypeStruct((NC, NHALF, D), jnp.float32),
    mesh=_MESH,
    scratch_types=[
        pltpu.VMEM((RPS, CH), jnp.int32),
        pltpu.VMEM((RPS, CH), jnp.int32),
        pltpu.VMEM((2, CH, D), jnp.float32),
        pltpu.VMEM_SHARED((ACC, D), jnp.float32),
        pltpu.SemaphoreType.DMA,
        pltpu.SemaphoreType.DMA,
    ],
)
def _agg_sc(g_hbm, src_hbm, dstl_hbm, zeros_hbm, out_hbm,
            src_v, dst_v, buf, acc, g0, g1):
    cid = lax.axis_index("c")
    sid = lax.axis_index("s")
    gsem = (g0, g1)
    pltpu.sync_copy(zeros_hbm, acc.at[pl.ds(sid * APS, APS)])
    pltpu.sync_copy(src_hbm.at[pl.ds(sid * RPS, RPS)], src_v)
    pltpu.sync_copy(dstl_hbm.at[cid, pl.ds(sid * RPS, RPS)], dst_v)
    plsc.subcore_barrier()

    pltpu.async_copy(g_hbm.at[src_v.at[0]], buf.at[0], gsem[0])

    def body(i, carry):
        c0 = 2 * i
        c1 = c0 + 1
        pltpu.async_copy(g_hbm.at[src_v.at[c1]], buf.at[1], gsem[1])
        pltpu.make_async_copy(g_hbm.at[src_v.at[c0]], buf.at[0], gsem[0]).wait()
        pltpu.sync_copy(buf.at[0], acc.at[dst_v.at[c0]], add=True)

        @pl.when(c0 + 2 < RPS)
        def _():
            pltpu.async_copy(g_hbm.at[src_v.at[c0 + 2]], buf.at[0], gsem[0])

        pltpu.make_async_copy(g_hbm.at[src_v.at[c1]], buf.at[1], gsem[1]).wait()
        pltpu.sync_copy(buf.at[1], acc.at[dst_v.at[c1]], add=True)
        return carry

    lax.fori_loop(0, RPS // 2, body, 0)
    plsc.subcore_barrier()
    pltpu.sync_copy(acc.at[pl.ds(sid * (NHALF // NS), NHALF // NS)],
                    out_hbm.at[cid, pl.ds(sid * (NHALF // NS), NHALF // NS)])


BN = 1000


def _dinv_of(dp_ref):
    return lax.rsqrt(dp_ref[:, 0] + 1.0)


def _dstl_body(dst_ref, out_ref):
    d = dst_ref[...]
    for c in range(NC):
        base = c * NHALF
        inr = jnp.logical_and(d >= base, d < base + NHALF)
        out_ref[c] = jnp.where(inr, d - base, NHALF + (d & (TRASH - 1)))


def _dstl(dst):
    return pl.pallas_call(
        _dstl_body,
        in_specs=[pl.BlockSpec((ROWS, CH), lambda: (0, 0))],
        out_specs=pl.BlockSpec((NC, ROWS, CH), lambda: (0, 0, 0)),
        out_shape=jax.ShapeDtypeStruct((NC, ROWS, CH), jnp.int32),
    )(dst)


def _mm1_body(x_ref, w_ref, dp_ref, g_ref):
    dinv = _dinv_of(dp_ref)
    h = jnp.dot(x_ref[...], w_ref[...], preferred_element_type=jnp.float32)
    g_ref[...] = h * dinv[:, None]


def _mm2_body(s_ref, g_ref, dp_ref, b_ref, w_ref, out_ref):
    dinv = _dinv_of(dp_ref)
    s = s_ref[...] + g_ref[...]
    h1 = jnp.maximum(s * dinv[:, None] + b_ref[...], 0.0)
    h2 = jnp.dot(h1, w_ref[...], preferred_element_type=jnp.float32)
    out_ref[...] = h2 * dinv[:, None]


def _fin_body(s_ref, g_ref, dp_ref, b_ref, out_ref):
    dinv = _dinv_of(dp_ref)
    out_ref[...] = (s_ref[...] + g_ref[...]) * dinv[:, None] + b_ref[...]


def _mm1(x, w1, degp):
    return pl.pallas_call(
        _mm1_body,
        grid=(N // BN,),
        in_specs=[
            pl.BlockSpec((BN, D), lambda i: (i, 0)),
            pl.BlockSpec((D, H), lambda i: (0, 0)),
            pl.BlockSpec((BN, DW), lambda i: (i, 0)),
        ],
        out_specs=pl.BlockSpec((BN, H), lambda i: (i, 0)),
        out_shape=jax.ShapeDtypeStruct((N, H), jnp.float32),
    )(x, w1, degp)


def _mm2(s1, g1, degp, b1, w2):
    return pl.pallas_call(
        _mm2_body,
        grid=(N // BN,),
        in_specs=[
            pl.BlockSpec((BN, H), lambda i: (i, 0)),
            pl.BlockSpec((BN, H), lambda i: (i, 0)),
            pl.BlockSpec((BN, DW), lambda i: (i, 0)),
            pl.BlockSpec((1, H), lambda i: (0, 0)),
            pl.BlockSpec((H, H), lambda i: (0, 0)),
        ],
        out_specs=pl.BlockSpec((BN, H), lambda i: (i, 0)),
        out_shape=jax.ShapeDtypeStruct((N, H), jnp.float32),
    )(s1, g1, degp, b1, w2)


def _fin(s2, g2, degp, b2):
    return pl.pallas_call(
        _fin_body,
        grid=(N // BN,),
        in_specs=[
            pl.BlockSpec((BN, H), lambda i: (i, 0)),
            pl.BlockSpec((BN, H), lambda i: (i, 0)),
            pl.BlockSpec((BN, DW), lambda i: (i, 0)),
            pl.BlockSpec((1, H), lambda i: (0, 0)),
        ],
        out_specs=pl.BlockSpec((BN, H), lambda i: (i, 0)),
        out_shape=jax.ShapeDtypeStruct((N, H), jnp.float32),
    )(s2, g2, degp, b2)


def kernel(x, edge_index, W1, b1, W2, b2):
    ei = edge_index.astype(jnp.int32)
    pad = EPAD - E
    src = jnp.concatenate([ei[0], jnp.zeros((pad,), jnp.int32)])
    dst = jnp.concatenate([ei[1], jnp.full((pad,), N, jnp.int32)])
    src = src.reshape(ROWS, CH)
    dst = dst.reshape(ROWS, CH)
    ones_w = jnp.ones((CH, DW), jnp.float32)
    zeros_w = jnp.zeros((APS, DW), jnp.float32)
    zeros_rows = jnp.zeros((APS, D), jnp.float32)

    dstl = _dstl(dst)
    degp = _deg_sc(dstl, ones_w, zeros_w).reshape(NPAD, DW)[:N]
    g1 = _mm1(x, W1, degp)
    s1 = _agg_sc(g1, src, dstl, zeros_rows).reshape(NPAD, D)[:N]
    g2 = _mm2(s1, g1, degp, b1.reshape(1, H), W2)
    s2 = _agg_sc(g2, src, dstl, zeros_rows).reshape(NPAD, D)[:N]
    return _fin(s2, g2, degp, b2.reshape(1, H))
